# Initial kernel scaffold; baseline (speedup 1.0000x reference)
import jax
import jax.numpy as jnp
from jax.experimental import pallas as pl


def kernel(x, edge_index, W1, b1, W2, b2, Wfc, bfc, Wout, bout):
    raise NotImplementedError("write your pallas kernel here")


# speedup vs baseline: 24.6830x; 24.6830x over previous
import functools

import jax
import jax.numpy as jnp
from jax import lax
from jax.experimental import pallas as pl
from jax.experimental.pallas import tpu as pltpu
from jax.experimental.pallas import tpu_sc as plsc

N_NODES = 10000
N_PAD = 10240
N_EDGES = 320000

NC = 2
NS = 16
NJ = 80
CH = 125
STRIPE = N_PAD // NS

_MESH = plsc.VectorSubcoreMesh(core_axis_name="c", subcore_axis_name="s")


DW = 8


@functools.partial(
    pl.kernel,
    out_type=jax.ShapeDtypeStruct((NC, N_PAD, DW), jnp.float32),
    mesh=_MESH,
    scratch_types=[
        pltpu.VMEM((NJ, CH), jnp.int32),
        pltpu.VMEM((CH, DW), jnp.float32),
        pltpu.VMEM((STRIPE, DW), jnp.float32),
        pltpu.VMEM_SHARED((N_PAD, DW), jnp.float32),
    ],
    compiler_params=pltpu.CompilerParams(use_tc_tiling_on_sc=False),
)
def _deg_kernel(dst_hbm, ones_hbm, zeros_hbm, out_hbm, dstb, onesb, zb, deg_s):
    c = lax.axis_index("c")
    s = lax.axis_index("s")
    pltpu.sync_copy(dst_hbm.at[c, s], dstb)
    pltpu.sync_copy(ones_hbm, onesb)
    pltpu.sync_copy(zeros_hbm, zb)
    pltpu.sync_copy(zb, deg_s.at[pl.ds(s * STRIPE, STRIPE)])
    plsc.subcore_barrier()

    @pl.loop(0, NJ)
    def _(j):
        pltpu.sync_copy(onesb, deg_s.at[dstb.at[j]], add=True)

    plsc.subcore_barrier()
    pltpu.sync_copy(deg_s.at[pl.ds(s * STRIPE, STRIPE)],
                    out_hbm.at[c, pl.ds(s * STRIPE, STRIPE)])


def _make_mp_kernel(d_feat):
    @functools.partial(
        pl.kernel,
        out_type=jax.ShapeDtypeStruct((NC, N_PAD, d_feat), jnp.float32),
        mesh=_MESH,
        scratch_types=[
            pltpu.VMEM((NJ, CH), jnp.int32),
            pltpu.VMEM((NJ, CH), jnp.int32),
            pltpu.VMEM((CH, d_feat), jnp.float32),
            pltpu.VMEM((STRIPE, d_feat), jnp.float32),
            pltpu.VMEM_SHARED((N_PAD, d_feat), jnp.float32),
            pltpu.SemaphoreType.DMA,
        ],
        compiler_params=pltpu.CompilerParams(use_tc_tiling_on_sc=False),
    )
    def _mp(hp_hbm, src_hbm, dst_hbm, zeros_hbm, out_hbm,
            srcb, dstb, rows, zb, acc_s, sem):
        c = lax.axis_index("c")
        s = lax.axis_index("s")
        pltpu.sync_copy(src_hbm.at[c, s], srcb)
        pltpu.sync_copy(dst_hbm.at[c, s], dstb)
        pltpu.sync_copy(zeros_hbm, zb)
        pltpu.sync_copy(zb, acc_s.at[pl.ds(s * STRIPE, STRIPE)])
        plsc.subcore_barrier()

        @pl.loop(0, NJ)
        def _(j):
            pltpu.async_copy(hp_hbm.at[srcb.at[j]], rows, sem).wait()
            pltpu.sync_copy(rows, acc_s.at[dstb.at[j]], add=True)

        plsc.subcore_barrier()
        pltpu.sync_copy(acc_s.at[pl.ds(s * STRIPE, STRIPE)],
                        out_hbm.at[c, pl.ds(s * STRIPE, STRIPE)])

    return _mp


_mp64 = _make_mp_kernel(64)
_mp32 = _make_mp_kernel(32)


def _k1_body(x_ref, w_ref, deg_ref, h_ref, dinv_ref):
    dinv = lax.rsqrt(deg_ref[:, 0:1] + deg_ref[:, 1:2] + 1.0)
    dinv_ref[...] = dinv
    h = jnp.dot(x_ref[...], w_ref[...], preferred_element_type=jnp.float32)
    h_ref[...] = h * dinv


def _k2_body(acc_ref, hp_ref, dinv_ref, b1_ref, w2_ref, out_ref):
    dinv = dinv_ref[...]
    t = (acc_ref[0, :N_NODES] + acc_ref[1, :N_NODES]
         + hp_ref[...]) * dinv + b1_ref[...]
    g = jnp.maximum(t, 0.0)
    out_ref[...] = jnp.dot(g, w2_ref[...],
                           preferred_element_type=jnp.float32) * dinv


def _k3_body(acc_ref, hp_ref, dinv_ref, b2_ref, wfc_ref, bfc_ref,
             wout_ref, bout_ref, o_ref):
    dinv = dinv_ref[...]
    t = (acc_ref[0, :N_NODES] + acc_ref[1, :N_NODES]
         + hp_ref[...]) * dinv + b2_ref[...]
    g = jnp.maximum(t, 0.0)
    g = jnp.maximum(jnp.dot(g, wfc_ref[...],
                            preferred_element_type=jnp.float32) + bfc_ref[...],
                    0.0)
    o_ref[...] = jnp.dot(g, wout_ref[...],
                         preferred_element_type=jnp.float32) + bout_ref[...]


def kernel(x, edge_index, W1, b1, W2, b2, Wfc, bfc, Wout, bout):
    ei = edge_index.astype(jnp.int32)
    src = ei[0].reshape(NC, NS, NJ, CH)
    dst = ei[1].reshape(NC, NS, NJ, CH)

    ones_ch = jnp.ones((CH, DW), jnp.float32)
    zeros_n = jnp.zeros((STRIPE, DW), jnp.float32)
    zeros64 = jnp.zeros((STRIPE, 64), jnp.float32)
    zeros32 = jnp.zeros((STRIPE, 32), jnp.float32)

    deg = _deg_kernel(dst, ones_ch, zeros_n)
    deg_t = deg[:, :N_NODES, 0].T

    h1p, dinv = pl.pallas_call(
        _k1_body,
        out_shape=(
            jax.ShapeDtypeStruct((N_NODES, 64), jnp.float32),
            jax.ShapeDtypeStruct((N_NODES, 1), jnp.float32),
        ),
    )(x, W1, deg_t)

    acc1 = _mp64(h1p, src, dst, zeros64)

    h2p = pl.pallas_call(
        _k2_body,
        out_shape=jax.ShapeDtypeStruct((N_NODES, 32), jnp.float32),
    )(acc1, h1p, dinv, b1.reshape(1, 64), W2)

    acc2 = _mp32(h2p, src, dst, zeros32)

    out = pl.pallas_call(
        _k3_body,
        out_shape=jax.ShapeDtypeStruct((N_NODES, 2), jnp.float32),
    )(acc2, h2p, dinv, b2.reshape(1, 32), Wfc, bfc.reshape(1, 32),
      Wout, bout.reshape(1, 2))
    return out
